# baseline (speedup 1.0000x reference)
import jax
import jax.numpy as jnp
from jax import lax
from jax.experimental import pallas as pl
from jax.experimental.pallas import tpu as pltpu
from jax.experimental.pallas import tpu_sc as plsc

N = 10000
E = 320000
D = 128
NC = 2
NS = 16
NW = NC * NS
K = 128
NCHUNK = 80
NSTG = 5
CPST = NCHUNK // NSTG
EPAD = NW * NCHUNK * K
NT = N + 16
RPS = 640
NPAD = NS * RPS

_mesh = plsc.VectorSubcoreMesh(core_axis_name="c", subcore_axis_name="s")


ZR = 20


def _zero_stripe(zbuf_v, shared, sid):
    def _fill(r, _):
        for cc in range(D // 16):
            zbuf_v[r, pl.ds(cc * 16, 16)] = jnp.zeros((16,), jnp.float32)
        return 0

    lax.fori_loop(0, ZR, _fill, 0)

    def _copy(i, _):
        pltpu.sync_copy(zbuf_v, shared.at[pl.ds(sid * RPS + i * ZR, ZR)])
        return 0

    lax.fori_loop(0, RPS // ZR, _copy, 0)


def _deg_body(dst_hbm, deg_hbm, dst_v, ones_v, zbuf_v, acc_shared):
    cid = lax.axis_index("c")
    sid = lax.axis_index("s")
    wid = cid * NS + sid

    def _fill_ones(r, _):
        for cc in range(D // 16):
            ones_v[r, pl.ds(cc * 16, 16)] = jnp.ones((16,), jnp.float32)
        return 0

    lax.fori_loop(0, K, _fill_ones, 0)
    _zero_stripe(zbuf_v, acc_shared, sid)
    plsc.subcore_barrier()

    pltpu.sync_copy(dst_hbm.at[wid], dst_v)

    def _chunk(j, _):
        pltpu.sync_copy(ones_v, acc_shared.at[dst_v.at[j]], add=True)
        return 0

    lax.fori_loop(0, NCHUNK, _chunk, 0)
    plsc.subcore_barrier()
    pltpu.sync_copy(acc_shared.at[pl.ds(sid * RPS, RPS)],
                    deg_hbm.at[cid, pl.ds(sid * RPS, RPS)])


def _scatter_body(h_hbm, src_hbm, dst_hbm, acc_hbm,
                  src_v, dst_v, rows0_v, zbuf_v, acc_shared, sem0):
    cid = lax.axis_index("c")
    sid = lax.axis_index("s")
    wid = cid * NS + sid

    pltpu.sync_copy(src_hbm.at[wid], src_v)
    pltpu.sync_copy(dst_hbm.at[wid], dst_v)
    _zero_stripe(zbuf_v, acc_shared, sid)
    plsc.subcore_barrier()

    def _chunk(j, _):
        pltpu.async_copy(h_hbm.at[src_v.at[j]], rows0_v, sem0).wait()
        pltpu.sync_copy(rows0_v, acc_shared.at[dst_v.at[j]], add=True)
        return 0

    lax.fori_loop(0, NCHUNK, _chunk, 0)
    plsc.subcore_barrier()
    pltpu.sync_copy(acc_shared.at[pl.ds(sid * RPS, RPS)],
                    acc_hbm.at[cid, pl.ds(sid * RPS, RPS)])


def _make_deg_kernel():
    return pl.kernel(
        _deg_body,
        out_type=jax.ShapeDtypeStruct((NC, NPAD, D), jnp.float32),
        mesh=_mesh,
        scratch_types=[
            pltpu.VMEM((NCHUNK, K), jnp.int32),
            pltpu.VMEM((K, D), jnp.float32),
            pltpu.VMEM((ZR, D), jnp.float32),
            pltpu.VMEM_SHARED((NPAD, D), jnp.float32),
        ],
    )


def _make_scatter_kernel():
    return pl.kernel(
        _scatter_body,
        out_type=jax.ShapeDtypeStruct((NC, NPAD, D), jnp.float32),
        mesh=_mesh,
        scratch_types=[
            pltpu.VMEM((NCHUNK, K), jnp.int32),
            pltpu.VMEM((NCHUNK, K), jnp.int32),
            pltpu.VMEM((K, D), jnp.float32),
            pltpu.VMEM((ZR, D), jnp.float32),
            pltpu.VMEM_SHARED((NPAD, D), jnp.float32),
            pltpu.SemaphoreType.DMA,
        ],
    )


def _tc_a_body(x_ref, w_ref, deg_ref, h_ref, dinv_ref):
    deg = deg_ref[0, :N, 0:1] + deg_ref[1, :N, 0:1] + 1.0
    dinv = lax.rsqrt(deg)
    dinv_ref[...] = dinv
    h = jnp.dot(x_ref[...], w_ref[...], preferred_element_type=jnp.float32)
    h_ref[:N, :] = h * dinv
    h_ref[N:, :] = jnp.zeros((NT - N, D), jnp.float32)


def _tc_b_body(acc_ref, hp_ref, dinv_ref, b_ref, w_ref, o_ref):
    dinv = dinv_ref[...]
    t = acc_ref[0, :N, :] + acc_ref[1, :N, :] + hp_ref[:N, :]
    y = jnp.maximum(t * dinv + b_ref[...], 0.0)
    o_ref[:N, :] = jnp.dot(y, w_ref[...], preferred_element_type=jnp.float32) * dinv
    o_ref[N:, :] = jnp.zeros((NT - N, D), jnp.float32)


def _tc_c_body(acc_ref, hp_ref, dinv_ref, b_ref, o_ref):
    t = acc_ref[0, :N, :] + acc_ref[1, :N, :] + hp_ref[:N, :]
    h = t * dinv_ref[...] + b_ref[...]
    nrm = jnp.sqrt(jnp.sum(h * h, axis=1, keepdims=True))
    o_ref[...] = h / jnp.maximum(nrm, 1e-12)


def _tc_call(body, out_shape, *args):
    return pl.pallas_call(body, out_shape=out_shape)(*args)


@jax.jit
def kernel(x, edge_index, W1, b1, W2, b2):
    ei = edge_index.astype(jnp.int32)
    pad = jnp.full((EPAD - E,), N, jnp.int32)
    src = jnp.concatenate([ei[0], pad]).reshape(NW, NCHUNK, K)
    dst = jnp.concatenate([ei[1], pad]).reshape(NW, NCHUNK, K)
    b1r = b1.reshape(1, D)
    b2r = b2.reshape(1, D)

    deg = _make_deg_kernel()(dst)
    h1p, dinv = _tc_call(
        _tc_a_body,
        (jax.ShapeDtypeStruct((NT, D), jnp.float32),
         jax.ShapeDtypeStruct((N, 1), jnp.float32)),
        x, W1, deg)
    acc1 = _make_scatter_kernel()(h1p, src, dst)
    h2p = _tc_call(
        _tc_b_body,
        jax.ShapeDtypeStruct((NT, D), jnp.float32),
        acc1, h1p, dinv, b1r, W2)
    acc2 = _make_scatter_kernel()(h2p, src, dst)
    out = _tc_call(
        _tc_c_body,
        jax.ShapeDtypeStruct((N, D), jnp.float32),
        acc2, h2p, dinv, b2r)
    return out


# speedup vs baseline: 2.4688x; 2.4688x over previous
import jax
import jax.numpy as jnp
from jax import lax
from jax.experimental import pallas as pl
from jax.experimental.pallas import tpu as pltpu
from jax.experimental.pallas import tpu_sc as plsc

N = 10000
E = 320000
D = 128
NC = 2
NS = 16
NW = NC * NS
K = 128
NCHUNK = 80
NSTG = 5
CPST = NCHUNK // NSTG
EPAD = NW * NCHUNK * K
NT = N + 16
RPS = 640
NPAD = NS * RPS

_mesh = plsc.VectorSubcoreMesh(core_axis_name="c", subcore_axis_name="s")


ZR = 20


def _zero_stripe(zbuf_v, shared, sid):
    def _fill(r, _):
        for cc in range(D // 16):
            zbuf_v[r, pl.ds(cc * 16, 16)] = jnp.zeros((16,), jnp.float32)
        return 0

    lax.fori_loop(0, ZR, _fill, 0)

    def _copy(i, _):
        pltpu.sync_copy(zbuf_v, shared.at[pl.ds(sid * RPS + i * ZR, ZR)])
        return 0

    lax.fori_loop(0, RPS // ZR, _copy, 0)


def _deg_body(dst_hbm, deg_hbm, dst_v, ones_v, zbuf_v, acc_shared):
    cid = lax.axis_index("c")
    sid = lax.axis_index("s")
    wid = cid * NS + sid

    def _fill_ones(r, _):
        for cc in range(D // 16):
            ones_v[r, pl.ds(cc * 16, 16)] = jnp.ones((16,), jnp.float32)
        return 0

    lax.fori_loop(0, K, _fill_ones, 0)
    _zero_stripe(zbuf_v, acc_shared, sid)
    plsc.subcore_barrier()

    pltpu.sync_copy(dst_hbm.at[wid], dst_v)

    def _chunk(j, _):
        pltpu.sync_copy(ones_v, acc_shared.at[dst_v.at[j]], add=True)
        return 0

    lax.fori_loop(0, NCHUNK, _chunk, 0)
    plsc.subcore_barrier()
    pltpu.sync_copy(acc_shared.at[pl.ds(sid * RPS, RPS)],
                    deg_hbm.at[cid, pl.ds(sid * RPS, RPS)])


def _scatter_body(h_hbm, src_hbm, dst_hbm, acc_hbm,
                  src_v, dst_v, rows0_v, zbuf_v, acc_shared, sem0):
    cid = lax.axis_index("c")
    sid = lax.axis_index("s")
    wid = cid * NS + sid

    pltpu.sync_copy(src_hbm.at[wid], src_v)
    pltpu.sync_copy(dst_hbm.at[wid], dst_v)
    _zero_stripe(zbuf_v, acc_shared, sid)
    plsc.subcore_barrier()

    def _chunk(j, _):
        pltpu.async_copy(h_hbm.at[src_v.at[j]], rows0_v, sem0).wait()
        pltpu.sync_copy(rows0_v, acc_shared.at[dst_v.at[j]], add=True)
        return 0

    lax.fori_loop(0, NCHUNK, _chunk, 0)
    plsc.subcore_barrier()
    pltpu.sync_copy(acc_shared.at[pl.ds(sid * RPS, RPS)],
                    acc_hbm.at[cid, pl.ds(sid * RPS, RPS)])


def _make_deg_kernel():
    return pl.kernel(
        _deg_body,
        out_type=jax.ShapeDtypeStruct((NC, NPAD, D), jnp.float32),
        mesh=_mesh,
        scratch_types=[
            pltpu.VMEM((NCHUNK, K), jnp.int32),
            pltpu.VMEM((K, D), jnp.float32),
            pltpu.VMEM((ZR, D), jnp.float32),
            pltpu.VMEM_SHARED((NPAD, D), jnp.float32),
        ],
    )


def _make_scatter_kernel():
    return pl.kernel(
        _scatter_body,
        out_type=jax.ShapeDtypeStruct((NC, NPAD, D), jnp.float32),
        mesh=_mesh,
        scratch_types=[
            pltpu.VMEM((NCHUNK, K), jnp.int32),
            pltpu.VMEM((NCHUNK, K), jnp.int32),
            pltpu.VMEM((K, D), jnp.float32),
            pltpu.VMEM((ZR, D), jnp.float32),
            pltpu.VMEM_SHARED((NPAD, D), jnp.float32),
            pltpu.SemaphoreType.DMA,
        ],
    )


def _tc_a_body(x_ref, w_ref, deg_ref, h_ref, dinv_ref):
    deg = deg_ref[0, :N, 0:1] + deg_ref[1, :N, 0:1] + 1.0
    dinv = lax.rsqrt(deg)
    dinv_ref[...] = dinv
    h = jnp.dot(x_ref[...], w_ref[...], preferred_element_type=jnp.float32)
    h_ref[:N, :] = h * dinv
    h_ref[N:, :] = jnp.zeros((NT - N, D), jnp.float32)


def _tc_b_body(acc_ref, hp_ref, dinv_ref, b_ref, w_ref, o_ref):
    dinv = dinv_ref[...]
    t = acc_ref[0, :N, :] + acc_ref[1, :N, :] + hp_ref[:N, :]
    y = jnp.maximum(t * dinv + b_ref[...], 0.0)
    o_ref[:N, :] = jnp.dot(y, w_ref[...], preferred_element_type=jnp.float32) * dinv
    o_ref[N:, :] = jnp.zeros((NT - N, D), jnp.float32)


def _tc_c_body(acc_ref, hp_ref, dinv_ref, b_ref, o_ref):
    t = acc_ref[0, :N, :] + acc_ref[1, :N, :] + hp_ref[:N, :]
    h = t * dinv_ref[...] + b_ref[...]
    nrm = jnp.sqrt(jnp.sum(h * h, axis=1, keepdims=True))
    o_ref[...] = h / jnp.maximum(nrm, 1e-12)


def _tc_call(body, out_shape, *args):
    return pl.pallas_call(body, out_shape=out_shape)(*args)


@jax.jit
def kernel(x, edge_index, W1, b1, W2, b2):
    ei = edge_index.astype(jnp.int32)
    pad_ar = jnp.arange(EPAD - E, dtype=jnp.int32)
    pad_src = N + pad_ar % (NT - N)
    pad_dst = N + pad_ar % (NPAD - N)
    src = jnp.concatenate([ei[0], pad_src]).reshape(NW, NCHUNK, K)
    dst = jnp.concatenate([ei[1], pad_dst]).reshape(NW, NCHUNK, K)
    b1r = b1.reshape(1, D)
    b2r = b2.reshape(1, D)

    deg = _make_deg_kernel()(dst)
    h1p, dinv = _tc_call(
        _tc_a_body,
        (jax.ShapeDtypeStruct((NT, D), jnp.float32),
         jax.ShapeDtypeStruct((N, 1), jnp.float32)),
        x, W1, deg)
    acc1 = _make_scatter_kernel()(h1p, src, dst)
    h2p = _tc_call(
        _tc_b_body,
        jax.ShapeDtypeStruct((NT, D), jnp.float32),
        acc1, h1p, dinv, b1r, W2)
    acc2 = _make_scatter_kernel()(h2p, src, dst)
    out = _tc_call(
        _tc_c_body,
        jax.ShapeDtypeStruct((N, D), jnp.float32),
        acc2, h2p, dinv, b2r)
    return out


# speedup vs baseline: 2.9144x; 1.1805x over previous
import jax
import jax.numpy as jnp
from jax import lax
from jax.experimental import pallas as pl
from jax.experimental.pallas import tpu as pltpu
from jax.experimental.pallas import tpu_sc as plsc

N = 10000
E = 320000
D = 128
NC = 2
NS = 16
NW = NC * NS
K = 128
NCHUNK = 80
NSTG = 2
CPST = NCHUNK // NSTG
EPAD = NW * NCHUNK * K
NT = N + 16
RPS = 632
NPAD = NS * RPS

_mesh = plsc.VectorSubcoreMesh(core_axis_name="c", subcore_axis_name="s")


def _zero_stripe(zbuf_v, shared, sid):
    def _fill(r, _):
        for cc in range(D // 16):
            zbuf_v[r, pl.ds(cc * 16, 16)] = jnp.zeros((16,), jnp.float32)
        return 0

    lax.fori_loop(0, 40, _fill, 0)
    for i in range(15):
        pltpu.sync_copy(zbuf_v, shared.at[pl.ds(sid * RPS + i * 40, 40)])
    pltpu.sync_copy(zbuf_v.at[pl.ds(0, 32)],
                    shared.at[pl.ds(sid * RPS + 600, 32)])


def _deg_body(dst_hbm, deg_hbm, dst_v, ones_v, zbuf_v, acc_shared):
    cid = lax.axis_index("c")
    sid = lax.axis_index("s")
    wid = cid * NS + sid

    def _fill_ones(r, _):
        for cc in range(D // 16):
            ones_v[r, pl.ds(cc * 16, 16)] = jnp.ones((16,), jnp.float32)
        return 0

    lax.fori_loop(0, K, _fill_ones, 0)
    _zero_stripe(zbuf_v, acc_shared, sid)
    pltpu.sync_copy(dst_hbm.at[wid], dst_v)
    plsc.subcore_barrier()

    def _chunk(j, _):
        pltpu.sync_copy(ones_v, acc_shared.at[dst_v.at[j]], add=True)
        return 0

    lax.fori_loop(0, NCHUNK, _chunk, 0)
    plsc.subcore_barrier()
    pltpu.sync_copy(acc_shared.at[pl.ds(sid * RPS, RPS)],
                    deg_hbm.at[cid, pl.ds(sid * RPS, RPS)])


def _scatter_body(h_hbm, src_hbm, dst_hbm, acc_hbm,
                  src_v, dst_v, rows0_v, rows1_v, zbuf_v, acc_shared,
                  sem0, sem1):
    cid = lax.axis_index("c")
    sid = lax.axis_index("s")
    wid = cid * NS + sid

    _zero_stripe(zbuf_v, acc_shared, sid)
    plsc.subcore_barrier()

    def _stage(s, _):
        pltpu.sync_copy(src_hbm.at[wid, pl.ds(s * CPST, CPST)], src_v)
        pltpu.sync_copy(dst_hbm.at[wid, pl.ds(s * CPST, CPST)], dst_v)
        pltpu.async_copy(h_hbm.at[src_v.at[0]], rows0_v, sem0)

        def _pair(i, _):
            j0 = 2 * i
            j1 = 2 * i + 1
            pltpu.make_async_copy(h_hbm.at[src_v.at[j0]], rows0_v, sem0).wait()
            pltpu.async_copy(h_hbm.at[src_v.at[j1]], rows1_v, sem1)
            pltpu.sync_copy(rows0_v, acc_shared.at[dst_v.at[j0]], add=True)
            pltpu.make_async_copy(h_hbm.at[src_v.at[j1]], rows1_v, sem1).wait()

            @pl.when(i < CPST // 2 - 1)
            def _():
                pltpu.async_copy(h_hbm.at[src_v.at[j0 + 2]], rows0_v, sem0)

            pltpu.sync_copy(rows1_v, acc_shared.at[dst_v.at[j1]], add=True)
            return 0

        lax.fori_loop(0, CPST // 2, _pair, 0)
        return 0

    lax.fori_loop(0, NSTG, _stage, 0)
    plsc.subcore_barrier()
    pltpu.sync_copy(acc_shared.at[pl.ds(sid * RPS, RPS)],
                    acc_hbm.at[cid, pl.ds(sid * RPS, RPS)])


def _make_deg_kernel():
    return pl.kernel(
        _deg_body,
        out_type=jax.ShapeDtypeStruct((NC, NPAD, D), jnp.float32),
        mesh=_mesh,
        scratch_types=[
            pltpu.VMEM((NCHUNK, K), jnp.int32),
            pltpu.VMEM((K, D), jnp.float32),
            pltpu.VMEM((40, D), jnp.float32),
            pltpu.VMEM_SHARED((NPAD, D), jnp.float32),
        ],
    )


def _make_scatter_kernel():
    return pl.kernel(
        _scatter_body,
        out_type=jax.ShapeDtypeStruct((NC, NPAD, D), jnp.float32),
        mesh=_mesh,
        scratch_types=[
            pltpu.VMEM((CPST, K), jnp.int32),
            pltpu.VMEM((CPST, K), jnp.int32),
            pltpu.VMEM((K, D), jnp.float32),
            pltpu.VMEM((K, D), jnp.float32),
            pltpu.VMEM((40, D), jnp.float32),
            pltpu.VMEM_SHARED((NPAD, D), jnp.float32),
            pltpu.SemaphoreType.DMA,
            pltpu.SemaphoreType.DMA,
        ],
    )


def _tc_a_body(x_ref, w_ref, deg_ref, h_ref, dinv_ref):
    deg = deg_ref[0, :N, 0:1] + deg_ref[1, :N, 0:1] + 1.0
    dinv = lax.rsqrt(deg)
    dinv_ref[...] = dinv
    h = jnp.dot(x_ref[...], w_ref[...], preferred_element_type=jnp.float32)
    h_ref[:N, :] = h * dinv
    h_ref[N:, :] = jnp.zeros((NT - N, D), jnp.float32)


def _tc_b_body(acc_ref, hp_ref, dinv_ref, b_ref, w_ref, o_ref):
    dinv = dinv_ref[...]
    t = acc_ref[0, :N, :] + acc_ref[1, :N, :] + hp_ref[:N, :]
    y = jnp.maximum(t * dinv + b_ref[...], 0.0)
    o_ref[:N, :] = jnp.dot(y, w_ref[...], preferred_element_type=jnp.float32) * dinv
    o_ref[N:, :] = jnp.zeros((NT - N, D), jnp.float32)


def _tc_c_body(acc_ref, hp_ref, dinv_ref, b_ref, o_ref):
    t = acc_ref[0, :N, :] + acc_ref[1, :N, :] + hp_ref[:N, :]
    h = t * dinv_ref[...] + b_ref[...]
    nrm = jnp.sqrt(jnp.sum(h * h, axis=1, keepdims=True))
    o_ref[...] = h / jnp.maximum(nrm, 1e-12)


def _tc_call(body, out_shape, *args):
    return pl.pallas_call(body, out_shape=out_shape)(*args)


@jax.jit
def kernel(x, edge_index, W1, b1, W2, b2):
    ei = edge_index.astype(jnp.int32)
    pad_ar = jnp.arange(EPAD - E, dtype=jnp.int32)
    pad_src = N + pad_ar % (NT - N)
    pad_dst = N + pad_ar % (NPAD - N)
    src = jnp.concatenate([ei[0], pad_src]).reshape(NW, NCHUNK, K)
    dst = jnp.concatenate([ei[1], pad_dst]).reshape(NW, NCHUNK, K)
    b1r = b1.reshape(1, D)
    b2r = b2.reshape(1, D)

    deg = _make_deg_kernel()(dst)
    h1p, dinv = _tc_call(
        _tc_a_body,
        (jax.ShapeDtypeStruct((NT, D), jnp.float32),
         jax.ShapeDtypeStruct((N, 1), jnp.float32)),
        x, W1, deg)
    acc1 = _make_scatter_kernel()(h1p, src, dst)
    h2p = _tc_call(
        _tc_b_body,
        jax.ShapeDtypeStruct((NT, D), jnp.float32),
        acc1, h1p, dinv, b1r, W2)
    acc2 = _make_scatter_kernel()(h2p, src, dst)
    out = _tc_call(
        _tc_c_body,
        jax.ShapeDtypeStruct((N, D), jnp.float32),
        acc2, h2p, dinv, b2r)
    return out


# speedup vs baseline: 3.1765x; 1.0899x over previous
import jax
import jax.numpy as jnp
from jax import lax
from jax.experimental import pallas as pl
from jax.experimental.pallas import tpu as pltpu
from jax.experimental.pallas import tpu_sc as plsc

N = 10000
E = 320000
D = 128
NC = 2
NS = 16
NW = NC * NS
K = 128
NCHUNK = 80
NSTG = 2
CPST = NCHUNK // NSTG
EPAD = NW * NCHUNK * K
NT = N + 16
RPS = 632
NPAD = NS * RPS

_mesh = plsc.VectorSubcoreMesh(core_axis_name="c", subcore_axis_name="s")


def _zero_stripe(zbuf_v, shared, sid):
    def _fill(r, _):
        for cc in range(D // 16):
            zbuf_v[r, pl.ds(cc * 16, 16)] = jnp.zeros((16,), jnp.float32)
        return 0

    lax.fori_loop(0, 40, _fill, 0)
    for i in range(15):
        pltpu.sync_copy(zbuf_v, shared.at[pl.ds(sid * RPS + i * 40, 40)])
    pltpu.sync_copy(zbuf_v.at[pl.ds(0, 32)],
                    shared.at[pl.ds(sid * RPS + 600, 32)])


def _deg_body(dst_hbm, deg_hbm, dst_v, ones_v, zbuf_v, acc_shared):
    cid = lax.axis_index("c")
    sid = lax.axis_index("s")
    wid = cid * NS + sid

    def _fill_ones(r, _):
        for cc in range(D // 16):
            ones_v[r, pl.ds(cc * 16, 16)] = jnp.ones((16,), jnp.float32)
        return 0

    lax.fori_loop(0, K, _fill_ones, 0)
    _zero_stripe(zbuf_v, acc_shared, sid)
    pltpu.sync_copy(dst_hbm.at[wid], dst_v)
    plsc.subcore_barrier()

    def _chunk(j, _):
        pltpu.sync_copy(ones_v, acc_shared.at[dst_v.at[j]], add=True)
        return 0

    lax.fori_loop(0, NCHUNK, _chunk, 0)
    plsc.subcore_barrier()
    pltpu.sync_copy(acc_shared.at[pl.ds(sid * RPS, RPS)],
                    deg_hbm.at[cid, pl.ds(sid * RPS, RPS)])


def _scatter_body(h_hbm, src_hbm, dst_hbm, acc_hbm,
                  src_v, dst_v, rows0_v, rows1_v, zbuf_v, acc_shared,
                  sem0, sem1):
    cid = lax.axis_index("c")
    sid = lax.axis_index("s")
    wid = cid * NS + sid

    _zero_stripe(zbuf_v, acc_shared, sid)
    plsc.subcore_barrier()

    def _stage(s, _):
        pltpu.sync_copy(src_hbm.at[wid, pl.ds(s * CPST, CPST)], src_v)
        pltpu.sync_copy(dst_hbm.at[wid, pl.ds(s * CPST, CPST)], dst_v)
        pltpu.async_copy(h_hbm.at[src_v.at[0]], rows0_v, sem0)

        def _pair(i, _):
            j0 = 2 * i
            j1 = 2 * i + 1
            pltpu.make_async_copy(h_hbm.at[src_v.at[j0]], rows0_v, sem0).wait()
            pltpu.async_copy(h_hbm.at[src_v.at[j1]], rows1_v, sem1)
            pltpu.sync_copy(rows0_v, acc_shared.at[dst_v.at[j0]], add=True)
            pltpu.make_async_copy(h_hbm.at[src_v.at[j1]], rows1_v, sem1).wait()

            @pl.when(i < CPST // 2 - 1)
            def _():
                pltpu.async_copy(h_hbm.at[src_v.at[j0 + 2]], rows0_v, sem0)

            pltpu.sync_copy(rows1_v, acc_shared.at[dst_v.at[j1]], add=True)
            return 0

        lax.fori_loop(0, CPST // 2, _pair, 0)
        return 0

    lax.fori_loop(0, NSTG, _stage, 0)
    plsc.subcore_barrier()
    pltpu.sync_copy(acc_shared.at[pl.ds(sid * RPS, RPS)],
                    acc_hbm.at[cid, pl.ds(sid * RPS, RPS)])


def _make_deg_kernel():
    return pl.kernel(
        _deg_body,
        out_type=jax.ShapeDtypeStruct((NC, NPAD, D), jnp.float32),
        mesh=_mesh,
        scratch_types=[
            pltpu.VMEM((NCHUNK, K), jnp.int32),
            pltpu.VMEM((K, D), jnp.float32),
            pltpu.VMEM((40, D), jnp.float32),
            pltpu.VMEM_SHARED((NPAD, D), jnp.float32),
        ],
    )


def _make_scatter_kernel():
    return pl.kernel(
        _scatter_body,
        out_type=jax.ShapeDtypeStruct((NC, NPAD, D), jnp.float32),
        mesh=_mesh,
        scratch_types=[
            pltpu.VMEM((CPST, K), jnp.int32),
            pltpu.VMEM((CPST, K), jnp.int32),
            pltpu.VMEM((K, D), jnp.float32),
            pltpu.VMEM((K, D), jnp.float32),
            pltpu.VMEM((40, D), jnp.float32),
            pltpu.VMEM_SHARED((NPAD, D), jnp.float32),
            pltpu.SemaphoreType.DMA,
            pltpu.SemaphoreType.DMA,
        ],
    )


def _tc_mm_body(x_ref, w_ref, h_ref):
    h_ref[...] = jnp.dot(x_ref[...], w_ref[...],
                         preferred_element_type=jnp.float32)


def _tc_a_body(h1_ref, deg_ref, h_ref, dinv_ref):
    deg = deg_ref[0, :N, 0:1] + deg_ref[1, :N, 0:1] + 1.0
    dinv = lax.rsqrt(deg)
    dinv_ref[...] = dinv
    h_ref[:N, :] = h1_ref[...] * dinv
    h_ref[N:, :] = jnp.zeros((NT - N, D), jnp.float32)


def _tc_b_body(acc_ref, hp_ref, dinv_ref, b_ref, w_ref, o_ref):
    dinv = dinv_ref[...]
    t = acc_ref[0, :N, :] + acc_ref[1, :N, :] + hp_ref[:N, :]
    y = jnp.maximum(t * dinv + b_ref[...], 0.0)
    o_ref[:N, :] = jnp.dot(y, w_ref[...], preferred_element_type=jnp.float32) * dinv
    o_ref[N:, :] = jnp.zeros((NT - N, D), jnp.float32)


def _tc_c_body(acc_ref, hp_ref, dinv_ref, b_ref, o_ref):
    t = acc_ref[0, :N, :] + acc_ref[1, :N, :] + hp_ref[:N, :]
    h = t * dinv_ref[...] + b_ref[...]
    nrm = jnp.sqrt(jnp.sum(h * h, axis=1, keepdims=True))
    o_ref[...] = h / jnp.maximum(nrm, 1e-12)


def _tc_call(body, out_shape, *args):
    return pl.pallas_call(body, out_shape=out_shape)(*args)


@jax.jit
def kernel(x, edge_index, W1, b1, W2, b2):
    ei = edge_index.astype(jnp.int32)
    pad_ar = jnp.arange(EPAD - E, dtype=jnp.int32)
    pad_src = N + pad_ar % (NT - N)
    pad_dst = N + pad_ar % (NPAD - N)
    src = jnp.concatenate([ei[0], pad_src]).reshape(NW, NCHUNK, K)
    dst = jnp.concatenate([ei[1], pad_dst]).reshape(NW, NCHUNK, K)
    b1r = b1.reshape(1, D)
    b2r = b2.reshape(1, D)

    deg = _make_deg_kernel()(dst)
    h1 = _tc_call(
        _tc_mm_body,
        jax.ShapeDtypeStruct((N, D), jnp.float32),
        x, W1)
    h1p, dinv = _tc_call(
        _tc_a_body,
        (jax.ShapeDtypeStruct((NT, D), jnp.float32),
         jax.ShapeDtypeStruct((N, 1), jnp.float32)),
        h1, deg)
    acc1 = _make_scatter_kernel()(h1p, src, dst)
    h2p = _tc_call(
        _tc_b_body,
        jax.ShapeDtypeStruct((NT, D), jnp.float32),
        acc1, h1p, dinv, b1r, W2)
    acc2 = _make_scatter_kernel()(h2p, src, dst)
    out = _tc_call(
        _tc_c_body,
        jax.ShapeDtypeStruct((N, D), jnp.float32),
        acc2, h2p, dinv, b2r)
    return out


# speedup vs baseline: 3.1972x; 1.0065x over previous
import jax
import jax.numpy as jnp
from jax import lax
from jax.experimental import pallas as pl
from jax.experimental.pallas import tpu as pltpu
from jax.experimental.pallas import tpu_sc as plsc

N = 10000
E = 320000
D = 128
NC = 2
NS = 16
NW = NC * NS
K = 128
NCHUNK = 80
NSTG = 2
CPST = NCHUNK // NSTG
EPAD = NW * NCHUNK * K
NT = N + 16
RPS = 632
NPAD = NS * RPS

_mesh = plsc.VectorSubcoreMesh(core_axis_name="c", subcore_axis_name="s")


def _zero_stripe(zbuf_v, shared, sid):
    def _fill(r, _):
        for cc in range(D // 16):
            zbuf_v[r, pl.ds(cc * 16, 16)] = jnp.zeros((16,), jnp.float32)
        return 0

    lax.fori_loop(0, 40, _fill, 0)
    for i in range(15):
        pltpu.sync_copy(zbuf_v, shared.at[pl.ds(sid * RPS + i * 40, 40)])
    pltpu.sync_copy(zbuf_v.at[pl.ds(0, 32)],
                    shared.at[pl.ds(sid * RPS + 600, 32)])


def _deg_body(dst_hbm, deg_hbm, dst_v, ones_v, zbuf_v, acc_shared):
    cid = lax.axis_index("c")
    sid = lax.axis_index("s")
    wid = cid * NS + sid

    def _fill_ones(r, _):
        for cc in range(D // 16):
            ones_v[r, pl.ds(cc * 16, 16)] = jnp.ones((16,), jnp.float32)
        return 0

    lax.fori_loop(0, K, _fill_ones, 0)
    _zero_stripe(zbuf_v, acc_shared, sid)
    pltpu.sync_copy(dst_hbm.at[wid], dst_v)
    plsc.subcore_barrier()

    def _chunk(j, _):
        pltpu.sync_copy(ones_v, acc_shared.at[dst_v.at[j]], add=True)
        return 0

    lax.fori_loop(0, NCHUNK, _chunk, 0)
    plsc.subcore_barrier()
    pltpu.sync_copy(acc_shared.at[pl.ds(sid * RPS, RPS)],
                    deg_hbm.at[cid, pl.ds(sid * RPS, RPS)])


def _scatter_body(h_hbm, src_hbm, dst_hbm, acc_hbm,
                  src_v, dst_v, rows0_v, rows1_v, zbuf_v, acc_shared,
                  sem0, sem1):
    cid = lax.axis_index("c")
    sid = lax.axis_index("s")
    wid = cid * NS + sid

    def _fill(r, _):
        for cc in range(D // 16):
            zbuf_v[r, pl.ds(cc * 16, 16)] = jnp.zeros((16,), jnp.float32)
        return 0

    lax.fori_loop(0, 40, _fill, 0)
    for i in range(15):
        pltpu.async_copy(zbuf_v, acc_shared.at[pl.ds(sid * RPS + i * 40, 40)],
                         sem1)
    pltpu.async_copy(zbuf_v.at[pl.ds(0, 32)],
                     acc_shared.at[pl.ds(sid * RPS + 600, 32)], sem1)
    pltpu.sync_copy(src_hbm.at[wid, pl.ds(0, CPST)], src_v)
    pltpu.sync_copy(dst_hbm.at[wid, pl.ds(0, CPST)], dst_v)
    for i in range(15):
        pltpu.make_async_copy(
            zbuf_v, acc_shared.at[pl.ds(sid * RPS + i * 40, 40)], sem1).wait()
    pltpu.make_async_copy(
        zbuf_v.at[pl.ds(0, 32)],
        acc_shared.at[pl.ds(sid * RPS + 600, 32)], sem1).wait()
    plsc.subcore_barrier()

    def _stage(s, _):
        @pl.when(s > 0)
        def _():
            pltpu.sync_copy(src_hbm.at[wid, pl.ds(s * CPST, CPST)], src_v)
            pltpu.sync_copy(dst_hbm.at[wid, pl.ds(s * CPST, CPST)], dst_v)
        pltpu.async_copy(h_hbm.at[src_v.at[0]], rows0_v, sem0)

        def _pair(i, _):
            j0 = 2 * i
            j1 = 2 * i + 1
            pltpu.make_async_copy(h_hbm.at[src_v.at[j0]], rows0_v, sem0).wait()
            pltpu.async_copy(h_hbm.at[src_v.at[j1]], rows1_v, sem1)
            pltpu.sync_copy(rows0_v, acc_shared.at[dst_v.at[j0]], add=True)
            pltpu.make_async_copy(h_hbm.at[src_v.at[j1]], rows1_v, sem1).wait()

            @pl.when(i < CPST // 2 - 1)
            def _():
                pltpu.async_copy(h_hbm.at[src_v.at[j0 + 2]], rows0_v, sem0)

            pltpu.sync_copy(rows1_v, acc_shared.at[dst_v.at[j1]], add=True)
            return 0

        lax.fori_loop(0, CPST // 2, _pair, 0)
        return 0

    lax.fori_loop(0, NSTG, _stage, 0)
    plsc.subcore_barrier()
    pltpu.sync_copy(acc_shared.at[pl.ds(sid * RPS, RPS)],
                    acc_hbm.at[cid, pl.ds(sid * RPS, RPS)])


def _make_deg_kernel():
    return pl.kernel(
        _deg_body,
        out_type=jax.ShapeDtypeStruct((NC, NPAD, D), jnp.float32),
        mesh=_mesh,
        scratch_types=[
            pltpu.VMEM((NCHUNK, K), jnp.int32),
            pltpu.VMEM((K, D), jnp.float32),
            pltpu.VMEM((40, D), jnp.float32),
            pltpu.VMEM_SHARED((NPAD, D), jnp.float32),
        ],
    )


def _make_scatter_kernel():
    return pl.kernel(
        _scatter_body,
        out_type=jax.ShapeDtypeStruct((NC, NPAD, D), jnp.float32),
        mesh=_mesh,
        scratch_types=[
            pltpu.VMEM((CPST, K), jnp.int32),
            pltpu.VMEM((CPST, K), jnp.int32),
            pltpu.VMEM((K, D), jnp.float32),
            pltpu.VMEM((K, D), jnp.float32),
            pltpu.VMEM((40, D), jnp.float32),
            pltpu.VMEM_SHARED((NPAD, D), jnp.float32),
            pltpu.SemaphoreType.DMA,
            pltpu.SemaphoreType.DMA,
        ],
    )


def _tc_mm_body(x_ref, w_ref, h_ref):
    h_ref[...] = jnp.dot(x_ref[...], w_ref[...],
                         preferred_element_type=jnp.float32)


def _tc_a_body(h1_ref, deg_ref, h_ref, dinv_ref):
    deg = deg_ref[0, :N, 0:1] + deg_ref[1, :N, 0:1] + 1.0
    dinv = lax.rsqrt(deg)
    dinv_ref[...] = dinv
    h_ref[:N, :] = h1_ref[...] * dinv
    h_ref[N:, :] = jnp.zeros((NT - N, D), jnp.float32)


def _tc_b_body(acc_ref, hp_ref, dinv_ref, b_ref, w_ref, o_ref):
    dinv = dinv_ref[...]
    t = acc_ref[0, :N, :] + acc_ref[1, :N, :] + hp_ref[:N, :]
    y = jnp.maximum(t * dinv + b_ref[...], 0.0)
    o_ref[:N, :] = jnp.dot(y, w_ref[...], preferred_element_type=jnp.float32) * dinv
    o_ref[N:, :] = jnp.zeros((NT - N, D), jnp.float32)


def _tc_c_body(acc_ref, hp_ref, dinv_ref, b_ref, o_ref):
    t = acc_ref[0, :N, :] + acc_ref[1, :N, :] + hp_ref[:N, :]
    h = t * dinv_ref[...] + b_ref[...]
    nrm = jnp.sqrt(jnp.sum(h * h, axis=1, keepdims=True))
    o_ref[...] = h / jnp.maximum(nrm, 1e-12)


def _tc_call(body, out_shape, *args):
    return pl.pallas_call(body, out_shape=out_shape)(*args)


@jax.jit
def kernel(x, edge_index, W1, b1, W2, b2):
    ei = edge_index.astype(jnp.int32)
    pad_ar = jnp.arange(EPAD - E, dtype=jnp.int32)
    pad_src = N + pad_ar % (NT - N)
    pad_dst = N + pad_ar % (NPAD - N)
    src = jnp.concatenate([ei[0], pad_src]).reshape(NW, NCHUNK, K)
    dst = jnp.concatenate([ei[1], pad_dst]).reshape(NW, NCHUNK, K)
    b1r = b1.reshape(1, D)
    b2r = b2.reshape(1, D)

    deg = _make_deg_kernel()(dst)
    h1 = _tc_call(
        _tc_mm_body,
        jax.ShapeDtypeStruct((N, D), jnp.float32),
        x, W1)
    h1p, dinv = _tc_call(
        _tc_a_body,
        (jax.ShapeDtypeStruct((NT, D), jnp.float32),
         jax.ShapeDtypeStruct((N, 1), jnp.float32)),
        h1, deg)
    acc1 = _make_scatter_kernel()(h1p, src, dst)
    h2p = _tc_call(
        _tc_b_body,
        jax.ShapeDtypeStruct((NT, D), jnp.float32),
        acc1, h1p, dinv, b1r, W2)
    acc2 = _make_scatter_kernel()(h2p, src, dst)
    out = _tc_call(
        _tc_c_body,
        jax.ShapeDtypeStruct((N, D), jnp.float32),
        acc2, h2p, dinv, b2r)
    return out


# speedup vs baseline: 3.2158x; 1.0058x over previous
import jax
import jax.numpy as jnp
from jax import lax
from jax.experimental import pallas as pl
from jax.experimental.pallas import tpu as pltpu
from jax.experimental.pallas import tpu_sc as plsc

N = 10000
E = 320000
D = 128
NC = 2
NS = 16
NW = NC * NS
K = 128
NCHUNK = 80
NSTG = 2
CPST = NCHUNK // NSTG
EPAD = NW * NCHUNK * K
NT = N + 16
RPS = 632
NPAD = NS * RPS

_mesh = plsc.VectorSubcoreMesh(core_axis_name="c", subcore_axis_name="s")


def _deg_body(dst_hbm, deg_hbm, dst_v, ones_v, zbuf_v, acc_shared,
              sem0, sem1):
    cid = lax.axis_index("c")
    sid = lax.axis_index("s")
    wid = cid * NS + sid

    def _fill_ones(r, _):
        for cc in range(D // 16):
            ones_v[r, pl.ds(cc * 16, 16)] = jnp.ones((16,), jnp.float32)
        return 0

    lax.fori_loop(0, K, _fill_ones, 0)

    def _fill(r, _):
        for cc in range(D // 16):
            zbuf_v[r, pl.ds(cc * 16, 16)] = jnp.zeros((16,), jnp.float32)
        return 0

    lax.fori_loop(0, 40, _fill, 0)
    for i in range(15):
        pltpu.async_copy(zbuf_v, acc_shared.at[pl.ds(sid * RPS + i * 40, 40)],
                         sem1)
    pltpu.async_copy(zbuf_v.at[pl.ds(0, 32)],
                     acc_shared.at[pl.ds(sid * RPS + 600, 32)], sem1)
    pltpu.sync_copy(dst_hbm.at[wid], dst_v)
    for i in range(15):
        pltpu.make_async_copy(
            zbuf_v, acc_shared.at[pl.ds(sid * RPS + i * 40, 40)], sem1).wait()
    pltpu.make_async_copy(
        zbuf_v.at[pl.ds(0, 32)],
        acc_shared.at[pl.ds(sid * RPS + 600, 32)], sem1).wait()
    plsc.subcore_barrier()

    def _pair(i, _):
        j0 = 2 * i
        j1 = 2 * i + 1
        pltpu.async_copy(ones_v, acc_shared.at[dst_v.at[j0]], sem0, add=True)
        pltpu.async_copy(ones_v, acc_shared.at[dst_v.at[j1]], sem1, add=True)
        pltpu.make_async_copy(ones_v, acc_shared.at[dst_v.at[j0]], sem0).wait()
        pltpu.make_async_copy(ones_v, acc_shared.at[dst_v.at[j1]], sem1).wait()
        return 0

    lax.fori_loop(0, NCHUNK // 2, _pair, 0)
    plsc.subcore_barrier()
    pltpu.sync_copy(acc_shared.at[pl.ds(sid * RPS, RPS)],
                    deg_hbm.at[cid, pl.ds(sid * RPS, RPS)])


def _scatter_body(h_hbm, src_hbm, dst_hbm, acc_hbm,
                  src_v, dst_v, rows0_v, rows1_v, zbuf_v, acc_shared,
                  sem0, sem1):
    cid = lax.axis_index("c")
    sid = lax.axis_index("s")
    wid = cid * NS + sid

    def _fill(r, _):
        for cc in range(D // 16):
            zbuf_v[r, pl.ds(cc * 16, 16)] = jnp.zeros((16,), jnp.float32)
        return 0

    lax.fori_loop(0, 40, _fill, 0)
    for i in range(15):
        pltpu.async_copy(zbuf_v, acc_shared.at[pl.ds(sid * RPS + i * 40, 40)],
                         sem1)
    pltpu.async_copy(zbuf_v.at[pl.ds(0, 32)],
                     acc_shared.at[pl.ds(sid * RPS + 600, 32)], sem1)
    pltpu.sync_copy(src_hbm.at[wid, pl.ds(0, CPST)], src_v)
    pltpu.sync_copy(dst_hbm.at[wid, pl.ds(0, CPST)], dst_v)
    for i in range(15):
        pltpu.make_async_copy(
            zbuf_v, acc_shared.at[pl.ds(sid * RPS + i * 40, 40)], sem1).wait()
    pltpu.make_async_copy(
        zbuf_v.at[pl.ds(0, 32)],
        acc_shared.at[pl.ds(sid * RPS + 600, 32)], sem1).wait()
    plsc.subcore_barrier()

    def _stage(s, _):
        @pl.when(s > 0)
        def _():
            pltpu.sync_copy(src_hbm.at[wid, pl.ds(s * CPST, CPST)], src_v)
            pltpu.sync_copy(dst_hbm.at[wid, pl.ds(s * CPST, CPST)], dst_v)
        pltpu.async_copy(h_hbm.at[src_v.at[0]], rows0_v, sem0)

        def _pair(i, _):
            j0 = 2 * i
            j1 = 2 * i + 1
            pltpu.make_async_copy(h_hbm.at[src_v.at[j0]], rows0_v, sem0).wait()
            pltpu.async_copy(h_hbm.at[src_v.at[j1]], rows1_v, sem1)
            pltpu.sync_copy(rows0_v, acc_shared.at[dst_v.at[j0]], add=True)
            pltpu.make_async_copy(h_hbm.at[src_v.at[j1]], rows1_v, sem1).wait()

            @pl.when(i < CPST // 2 - 1)
            def _():
                pltpu.async_copy(h_hbm.at[src_v.at[j0 + 2]], rows0_v, sem0)

            pltpu.sync_copy(rows1_v, acc_shared.at[dst_v.at[j1]], add=True)
            return 0

        lax.fori_loop(0, CPST // 2, _pair, 0)
        return 0

    lax.fori_loop(0, NSTG, _stage, 0)
    plsc.subcore_barrier()
    pltpu.sync_copy(acc_shared.at[pl.ds(sid * RPS, RPS)],
                    acc_hbm.at[cid, pl.ds(sid * RPS, RPS)])


def _make_deg_kernel():
    return pl.kernel(
        _deg_body,
        out_type=jax.ShapeDtypeStruct((NC, NPAD, D), jnp.float32),
        mesh=_mesh,
        scratch_types=[
            pltpu.VMEM((NCHUNK, K), jnp.int32),
            pltpu.VMEM((K, D), jnp.float32),
            pltpu.VMEM((40, D), jnp.float32),
            pltpu.VMEM_SHARED((NPAD, D), jnp.float32),
            pltpu.SemaphoreType.DMA,
            pltpu.SemaphoreType.DMA,
        ],
    )


def _make_scatter_kernel():
    return pl.kernel(
        _scatter_body,
        out_type=jax.ShapeDtypeStruct((NC, NPAD, D), jnp.float32),
        mesh=_mesh,
        scratch_types=[
            pltpu.VMEM((CPST, K), jnp.int32),
            pltpu.VMEM((CPST, K), jnp.int32),
            pltpu.VMEM((K, D), jnp.float32),
            pltpu.VMEM((K, D), jnp.float32),
            pltpu.VMEM((40, D), jnp.float32),
            pltpu.VMEM_SHARED((NPAD, D), jnp.float32),
            pltpu.SemaphoreType.DMA,
            pltpu.SemaphoreType.DMA,
        ],
    )


def _tc_mm_body(x_ref, w_ref, h_ref):
    h_ref[...] = jnp.dot(x_ref[...], w_ref[...],
                         preferred_element_type=jnp.float32)


def _tc_a_body(h1_ref, deg_ref, h_ref, dinv_ref):
    deg = deg_ref[0, :N, 0:1] + deg_ref[1, :N, 0:1] + 1.0
    dinv = lax.rsqrt(deg)
    dinv_ref[...] = dinv
    h_ref[:N, :] = h1_ref[...] * dinv
    h_ref[N:, :] = jnp.zeros((NT - N, D), jnp.float32)


def _tc_b_body(acc_ref, hp_ref, dinv_ref, b_ref, w_ref, o_ref):
    dinv = dinv_ref[...]
    t = acc_ref[0, :N, :] + acc_ref[1, :N, :] + hp_ref[:N, :]
    y = jnp.maximum(t * dinv + b_ref[...], 0.0)
    o_ref[:N, :] = jnp.dot(y, w_ref[...], preferred_element_type=jnp.float32) * dinv
    o_ref[N:, :] = jnp.zeros((NT - N, D), jnp.float32)


def _tc_c_body(acc_ref, hp_ref, dinv_ref, b_ref, o_ref):
    t = acc_ref[0, :N, :] + acc_ref[1, :N, :] + hp_ref[:N, :]
    h = t * dinv_ref[...] + b_ref[...]
    nrm = jnp.sqrt(jnp.sum(h * h, axis=1, keepdims=True))
    o_ref[...] = h / jnp.maximum(nrm, 1e-12)


def _tc_call(body, out_shape, *args):
    return pl.pallas_call(body, out_shape=out_shape)(*args)


@jax.jit
def kernel(x, edge_index, W1, b1, W2, b2):
    ei = edge_index.astype(jnp.int32)
    pad_ar = jnp.arange(EPAD - E, dtype=jnp.int32)
    pad_src = N + pad_ar % (NT - N)
    pad_dst = N + pad_ar % (NPAD - N)
    src = jnp.concatenate([ei[0], pad_src]).reshape(NW, NCHUNK, K)
    dst = jnp.concatenate([ei[1], pad_dst]).reshape(NW, NCHUNK, K)
    b1r = b1.reshape(1, D)
    b2r = b2.reshape(1, D)

    deg = _make_deg_kernel()(dst)
    h1 = _tc_call(
        _tc_mm_body,
        jax.ShapeDtypeStruct((N, D), jnp.float32),
        x, W1)
    h1p, dinv = _tc_call(
        _tc_a_body,
        (jax.ShapeDtypeStruct((NT, D), jnp.float32),
         jax.ShapeDtypeStruct((N, 1), jnp.float32)),
        h1, deg)
    acc1 = _make_scatter_kernel()(h1p, src, dst)
    h2p = _tc_call(
        _tc_b_body,
        jax.ShapeDtypeStruct((NT, D), jnp.float32),
        acc1, h1p, dinv, b1r, W2)
    acc2 = _make_scatter_kernel()(h2p, src, dst)
    out = _tc_call(
        _tc_c_body,
        jax.ShapeDtypeStruct((N, D), jnp.float32),
        acc2, h2p, dinv, b2r)
    return out
